# baseline (speedup 1.0000x reference)
import functools

import jax
import jax.numpy as jnp
from jax.experimental import pallas as pl
from jax.experimental.pallas import tpu as pltpu

_EPS = 1e-10
_N_TOK = 128
_VOCAB = 100000
_CHUNK = 40
_BLK_V = 20000
_NSTEP = _VOCAB // _BLK_V
_NCOL = _VOCAB // _CHUNK
_COLS_STEP = _BLK_V // _CHUNK
_BIG = 2**30


def _body(at_ref, invt_ref, gum_ref, out_ref, accv_ref, acci_ref):
    j = pl.program_id(0)

    @pl.when(j == 0)
    def _():
        accv_ref[...] = jnp.full((_CHUNK, _N_TOK), -jnp.inf, jnp.float32)
        acci_ref[...] = jnp.zeros((_CHUNK, _N_TOK), jnp.int32)

    invt = invt_ref[...]
    m = accv_ref[...]
    a = acci_ref[...]
    base0 = j * _BLK_V
    for q in range(_COLS_STEP):
        s = at_ref[q * _CHUNK:(q + 1) * _CHUNK, :] * invt + gum_ref[0, :, q:q + 1]
        upd = s > m
        m = jnp.where(upd, s, m)
        a = jnp.where(upd, base0 + q * _CHUNK, a)
    accv_ref[...] = m
    acci_ref[...] = a

    @pl.when(j == _NSTEP - 1)
    def _():
        best = m.max(axis=0, keepdims=True)
        sub = jax.lax.broadcasted_iota(jnp.int32, m.shape, 0)
        cand = a + sub
        out_ref[...] = jnp.min(
            jnp.where(m == best, cand, _BIG), axis=0, keepdims=True
        )


@functools.partial(jax.jit, static_argnames=())
def kernel(logits, temperatures, exponential):
    at = logits.T
    invt = (1.0 / jnp.clip(temperatures, _EPS, None)).reshape(1, _N_TOK)
    gum = (-jnp.log(exponential)).reshape(_NSTEP, _COLS_STEP, _CHUNK).transpose(0, 2, 1)
    out = pl.pallas_call(
        _body,
        grid=(_NSTEP,),
        in_specs=[
            pl.BlockSpec((_BLK_V, _N_TOK), lambda j: (j, 0)),
            pl.BlockSpec((1, _N_TOK), lambda j: (0, 0)),
            pl.BlockSpec((1, _CHUNK, _COLS_STEP), lambda j: (j, 0, 0)),
        ],
        out_specs=pl.BlockSpec((1, _N_TOK), lambda j: (0, 0)),
        out_shape=jax.ShapeDtypeStruct((1, _N_TOK), jnp.int32),
        scratch_shapes=[
            pltpu.VMEM((_CHUNK, _N_TOK), jnp.float32),
            pltpu.VMEM((_CHUNK, _N_TOK), jnp.int32),
        ],
    )(at, invt, gum)
    return out.reshape(_N_TOK)


# speedup vs baseline: 1.0480x; 1.0480x over previous
import functools

import jax
import jax.numpy as jnp
from jax.experimental import pallas as pl
from jax.experimental.pallas import tpu as pltpu

_EPS = 1e-10
_N_TOK = 128
_VOCAB = 100000
_CHUNK = 80
_BLK_V = 20000
_NSTEP = _VOCAB // _BLK_V
_NCOL = _VOCAB // _CHUNK
_COLS_STEP = _BLK_V // _CHUNK
_BIG = 2**30


def _body(at_ref, invt_ref, gum_ref, out_ref, accv_ref, acci_ref):
    j = pl.program_id(0)

    @pl.when(j == 0)
    def _():
        accv_ref[...] = jnp.full((_CHUNK, _N_TOK), -jnp.inf, jnp.float32)
        acci_ref[...] = jnp.zeros((_CHUNK, _N_TOK), jnp.int32)

    invt = invt_ref[...]
    gum = -jnp.log(gum_ref[0, :, :])
    m = accv_ref[...]
    a = acci_ref[...]
    base0 = j * _BLK_V
    for q in range(_COLS_STEP):
        s = at_ref[q * _CHUNK:(q + 1) * _CHUNK, :] * invt + gum[:, q:q + 1]
        upd = s > m
        m = jnp.where(upd, s, m)
        a = jnp.where(upd, base0 + q * _CHUNK, a)
    accv_ref[...] = m
    acci_ref[...] = a

    @pl.when(j == _NSTEP - 1)
    def _():
        best = m.max(axis=0, keepdims=True)
        sub = jax.lax.broadcasted_iota(jnp.int32, m.shape, 0)
        cand = a + sub
        out_ref[...] = jnp.min(
            jnp.where(m == best, cand, _BIG), axis=0, keepdims=True
        )


@functools.partial(jax.jit, static_argnames=())
def kernel(logits, temperatures, exponential):
    at = logits.T
    invt = (1.0 / jnp.clip(temperatures, _EPS, None)).reshape(1, _N_TOK)
    gum = exponential.reshape(_NSTEP, _COLS_STEP, _CHUNK).transpose(0, 2, 1)
    out = pl.pallas_call(
        _body,
        grid=(_NSTEP,),
        in_specs=[
            pl.BlockSpec((_BLK_V, _N_TOK), lambda j: (j, 0)),
            pl.BlockSpec((1, _N_TOK), lambda j: (0, 0)),
            pl.BlockSpec((1, _CHUNK, _COLS_STEP), lambda j: (j, 0, 0)),
        ],
        out_specs=pl.BlockSpec((1, _N_TOK), lambda j: (0, 0)),
        out_shape=jax.ShapeDtypeStruct((1, _N_TOK), jnp.int32),
        scratch_shapes=[
            pltpu.VMEM((_CHUNK, _N_TOK), jnp.float32),
            pltpu.VMEM((_CHUNK, _N_TOK), jnp.int32),
        ],
    )(at, invt, gum)
    return out.reshape(_N_TOK)


# speedup vs baseline: 1.0907x; 1.0408x over previous
import functools

import jax
import jax.numpy as jnp
from jax.experimental import pallas as pl
from jax.experimental.pallas import tpu as pltpu

_EPS = 1e-10
_N_TOK = 128
_VOCAB = 100000
_CHUNK = 80
_BLK_V = 20000
_NSTEP = _VOCAB // _BLK_V
_NCOL = _VOCAB // _CHUNK
_COLS_STEP = _BLK_V // _CHUNK
_BIG = 2**30


def _body(at_ref, invt_ref, gum_ref, out_ref, accv_ref, acci_ref):
    j = pl.program_id(0)

    @pl.when(j == 0)
    def _():
        accv_ref[...] = jnp.full((_CHUNK, _N_TOK), -jnp.inf, jnp.float32)
        acci_ref[...] = jnp.zeros((_CHUNK, _N_TOK), jnp.int32)

    invt = 1.0 / jnp.clip(invt_ref[...], _EPS, None)
    gum = -jnp.log(gum_ref[0, :, :])
    m = accv_ref[...]
    a = acci_ref[...]
    base0 = j * _BLK_V
    for q in range(_COLS_STEP):
        s = at_ref[q * _CHUNK:(q + 1) * _CHUNK, :] * invt + gum[:, q:q + 1]
        upd = s > m
        m = jnp.where(upd, s, m)
        a = jnp.where(upd, base0 + q * _CHUNK, a)
    accv_ref[...] = m
    acci_ref[...] = a

    @pl.when(j == _NSTEP - 1)
    def _():
        best = m.max(axis=0, keepdims=True)
        sub = jax.lax.broadcasted_iota(jnp.int32, m.shape, 0)
        cand = a + sub
        out_ref[...] = jnp.min(
            jnp.where(m == best, cand, _BIG), axis=0, keepdims=True
        )


@functools.partial(jax.jit, static_argnames=())
def kernel(logits, temperatures, exponential):
    at = logits.T
    invt = temperatures.reshape(1, _N_TOK)
    gum = exponential.reshape(_NSTEP, _COLS_STEP, _CHUNK).transpose(0, 2, 1)
    out = pl.pallas_call(
        _body,
        grid=(_NSTEP,),
        in_specs=[
            pl.BlockSpec((_BLK_V, _N_TOK), lambda j: (j, 0)),
            pl.BlockSpec((1, _N_TOK), lambda j: (0, 0)),
            pl.BlockSpec((1, _CHUNK, _COLS_STEP), lambda j: (j, 0, 0)),
        ],
        out_specs=pl.BlockSpec((1, _N_TOK), lambda j: (0, 0)),
        out_shape=jax.ShapeDtypeStruct((1, _N_TOK), jnp.int32),
        scratch_shapes=[
            pltpu.VMEM((_CHUNK, _N_TOK), jnp.float32),
            pltpu.VMEM((_CHUNK, _N_TOK), jnp.int32),
        ],
    )(at, invt, gum)
    return out.reshape(_N_TOK)
